# baseline (speedup 1.0000x reference)
import functools

import jax
import jax.numpy as jnp
from jax import lax
from jax.experimental import pallas as pl
from jax.experimental.pallas import tpu as pltpu
from jax.experimental.pallas import tpu_sc as plsc

_pc = pl.pallas_call

N = 10000
M = 1000
E0 = 320000
E1 = 16000
NP = 10240
MP = 1024
E0P = 327680
E1P = 16384
NC, NS = 2, 16
NW = NC * NS

def _mesh():
    return plsc.VectorSubcoreMesh(core_axis_name="c", subcore_axis_name="s")


_SC_PARAMS = pltpu.CompilerParams(use_tc_tiling_on_sc=False)
_SC_PARAMS_NL = pltpu.CompilerParams(use_tc_tiling_on_sc=False,
                                     needs_layout_passes=False)


f32 = jnp.float32
i32 = jnp.int32


def _sc_gather(n_t, n_rows, E, C, K, G):
    per_w = E // NW
    n_ch = per_w // C
    n_g = n_ch // G
    assert n_g * G == n_ch

    def body(*refs):
        tabs = refs[:n_t]
        idxs = refs[n_t:2 * n_t]
        outs = refs[2 * n_t:3 * n_t]
        idx_v, buf0, buf1, sem0, sem1 = refs[3 * n_t:]
        bufs = (buf0, buf1)
        sems = (sem0, sem1)
        wid = lax.axis_index("s") * NC + lax.axis_index("c")
        row0 = wid * n_ch

        for t in range(n_t):
            pltpu.sync_copy(idxs[t].at[pl.ds(row0, n_ch)], idx_v)

            def fire(g, b, t=t):
                return [pltpu.async_copy(
                    tabs[t].at[idx_v.at[g * G + i]],
                    bufs[b].at[i], sems[b]) for i in range(G)]

            pending = fire(0, 0)
            for g in range(n_g):
                b = g % 2
                nxt = fire(g + 1, 1 - b) if g + 1 < n_g else []
                for h in pending:
                    h.wait()
                pltpu.sync_copy(
                    bufs[b],
                    outs[t].at[pl.ds(row0 + g * G, G)])
                pending = nxt

    return pl.kernel(
        body,
        out_type=[jax.ShapeDtypeStruct((E // C, C, K), f32)
                  for _ in range(n_t)],
        mesh=_mesh(), compiler_params=_SC_PARAMS,
        name=f"scg_{n_t}x{E}x{K}",
        scratch_types=[
            pltpu.VMEM((n_ch, C), i32),
            pltpu.VMEM((G, C, K), f32),
            pltpu.VMEM((G, C, K), f32),
            pltpu.SemaphoreType.DMA,
            pltpu.SemaphoreType.DMA,
        ],
    )


def _sc_vgather2(E, CH):
    per_g = E // 4
    n_ch = per_g // CH

    def body(tab_a, idx_a, tab_b, idx_b, out_a, out_b,
             tslice, idx_v, ob0, ob1, sem0, sem1):
        c = lax.axis_index("c")
        s = lax.axis_index("s")
        wid = s * NC + c
        eg = wid // 8
        fs = wid % 8
        lanes = lax.iota(i32, 16)
        hi = lanes >> 3
        lo = lanes & 7
        obufs = (ob0, ob1)
        sems = (sem0, sem1)
        for tab, idx, out in ((tab_a, idx_a, out_a), (tab_b, idx_b, out_b)):
            pltpu.sync_copy(tab.at[fs], tslice)
            hby = {0: [], 1: []}
            for k in range(n_ch):
                b = k % 2
                for h in hby[b]:
                    h.wait()
                pltpu.sync_copy(idx.at[pl.ds(eg * per_g + k * CH, CH)],
                                idx_v)

                def step(j, _, b=b):
                    jv = jnp.full((16,), 2 * j, i32) + hi
                    ids = plsc.load_gather(idx_v, [jv])
                    val = plsc.load_gather(tslice, [ids, lo])
                    plsc.store_scatter(obufs[b], [jv, lo], val)
                    return 0

                lax.fori_loop(0, CH // 2, step, 0)
                hby[b] = [pltpu.async_copy(
                    obufs[b],
                    out.at[fs, pl.ds(eg * per_g + k * CH, CH)], sems[b])]
            for b in (0, 1):
                for h in hby[b]:
                    h.wait()

    return pl.kernel(
        body,
        out_type=[jax.ShapeDtypeStruct((8, E, 8), f32) for _ in range(2)],
        mesh=_mesh(), compiler_params=_SC_PARAMS_NL,
        name=f"scvg_{E}",
        scratch_types=[
            pltpu.VMEM((NP, 8), f32),
            pltpu.VMEM((CH,), i32),
            pltpu.VMEM((CH, 8), f32),
            pltpu.VMEM((CH, 8), f32),
            pltpu.SemaphoreType.DMA,
            pltpu.SemaphoreType.DMA,
        ],
    )


def _sc_scatter_add(E, C, Np, with_count, G):
    per_w = E // NW
    n_ch = per_w // C
    n_g = n_ch // G
    assert n_g * G == n_ch
    rpt = Np // NS

    def body(*refs):
        if with_count:
            (msg, idx, init, initc, ones, out, cnt_out,
             acc, cacc, idx_v, msg0, msg1, ones_v, sem0, sem1) = refs
        else:
            (msg, idx, init, out, acc, idx_v, msg0, msg1,
             sem0, sem1) = refs
        msgs = (msg0, msg1)
        sems = (sem0, sem1)
        c = lax.axis_index("c")
        s = lax.axis_index("s")
        wid = s * NC + c
        pltpu.sync_copy(init.at[c, pl.ds(s * rpt, rpt)],
                        acc.at[pl.ds(s * rpt, rpt)])
        if with_count:
            pltpu.sync_copy(initc.at[c, pl.ds(s * rpt, rpt)],
                            cacc.at[pl.ds(s * rpt, rpt)])
            pltpu.sync_copy(ones, ones_v)
        plsc.subcore_barrier()
        pltpu.sync_copy(idx.at[pl.ds(wid * n_ch, n_ch)], idx_v)

        def fire(g, b):
            hs = []
            for i in range(G):
                hs.append(pltpu.async_copy(
                    msgs[b].at[i], acc.at[idx_v.at[g * G + i]],
                    sems[b], add=True))
                if with_count:
                    hs.append(pltpu.async_copy(
                        ones_v.at[i], cacc.at[idx_v.at[g * G + i]],
                        sems[b], add=True))
            return hs

        pltpu.sync_copy(msg.at[pl.ds(wid * n_ch, G)], msg0)
        pending = fire(0, 0)
        for g in range(n_g):
            b = g % 2
            if g + 1 < n_g:
                pltpu.sync_copy(
                    msg.at[pl.ds(wid * n_ch + (g + 1) * G, G)],
                    msgs[1 - b])
                nxt = fire(g + 1, 1 - b)
            else:
                nxt = []
            for h in pending:
                h.wait()
            pending = nxt
        plsc.subcore_barrier()
        pltpu.sync_copy(acc.at[pl.ds(s * rpt, rpt)],
                        out.at[c, pl.ds(s * rpt, rpt)])
        if with_count:
            pltpu.sync_copy(cacc.at[pl.ds(s * rpt, rpt)],
                            cnt_out.at[c, pl.ds(s * rpt, rpt)])

    out_type = [jax.ShapeDtypeStruct((NC, Np, 64), f32)]
    scratch = [
        pltpu.VMEM_SHARED((Np, 64), f32),
    ]
    if with_count:
        out_type.append(jax.ShapeDtypeStruct((NC, Np, 16), f32))
        scratch.append(pltpu.VMEM_SHARED((Np, 16), f32))
    scratch += [pltpu.VMEM((n_ch, C), i32),
                pltpu.VMEM((G, C, 64), f32),
                pltpu.VMEM((G, C, 64), f32)]
    if with_count:
        scratch.append(pltpu.VMEM((G, C, 16), f32))
    scratch += [pltpu.SemaphoreType.DMA, pltpu.SemaphoreType.DMA]
    return pl.kernel(body, out_type=out_type, mesh=_mesh(),
                     compiler_params=_SC_PARAMS,
                     name=f"sca_{E}x{Np}" + ("_cnt" if with_count else ""),
                     scratch_types=scratch)


def _sc_scatter_max(E, Np, K):
    NEG = 8
    NFG = 4
    per_g = E // NEG
    CH = 512
    n_ch = per_g // CH

    def body(msg, idx, z64, out, acc, idx_v, buf):
        c = lax.axis_index("c")
        s = lax.axis_index("s")
        eg = c * (NS // NFG) + s // NFG
        fg = s % NFG
        lanes = [lax.iota(i32, 16) + 16 * q for q in range(4)]
        pltpu.sync_copy(z64, acc)
        pltpu.sync_copy(idx.at[pl.ds(eg * per_g, per_g)], idx_v)
        for k in range(n_ch):
            pltpu.sync_copy(
                msg.at[pl.ds(eg * per_g + k * CH, CH), pl.ds(fg * 64, 64)],
                buf)

            def step(e, _, k=k):
                dstv = plsc.load_gather(idx_v, [jnp.full((16,), k * CH + e, i32)])
                ev = jnp.full((16,), e, i32)
                for q in range(4):
                    val = plsc.load_gather(buf, [ev, lanes[q]])
                    cur = plsc.load_gather(acc, [dstv, lanes[q]])
                    plsc.store_scatter(acc, [dstv, lanes[q]],
                                       jnp.maximum(cur, val))
                return 0

            lax.fori_loop(0, CH, step, 0)
        pltpu.sync_copy(acc, out.at[eg, pl.ds(0, Np), pl.ds(fg * 64, 64)])

    return pl.kernel(
        body,
        out_type=[jax.ShapeDtypeStruct((NEG, Np, K), f32)],
        mesh=_mesh(), compiler_params=_SC_PARAMS_NL,
        name=f"scm_{E}x{Np}",
        scratch_types=[
            pltpu.VMEM((Np, 64), f32),
            pltpu.VMEM((per_g,), i32),
            pltpu.VMEM((CH, 64), f32),
        ],
    )


def _dot(a, b):
    return jnp.dot(a, b, preferred_element_type=f32)


def _rows(bs, w):
    return pl.BlockSpec((bs, w), lambda i: (i, 0))


def _full(shape):
    return pl.BlockSpec(shape, lambda i: tuple(0 for _ in shape))


def _tc_node_proj(feats, pts8, waf, wap, wcf, wcp, bc):
    K = waf.shape[1]
    bs = 512

    def body(xf, xp, rwaf, rwap, rwcf, rwcp, rbc, oa, oc):
        fb = xf[...]
        pb = xp[...]
        oa[...] = _dot(fb, rwaf[...]) + _dot(pb, rwap[...])
        oc[...] = _dot(fb, rwcf[...]) + _dot(pb, rwcp[...]) + rbc[...]

    return _pc(
        body,
        grid=(NP // bs,),
        in_specs=[_rows(bs, 128), _rows(bs, 8), _full(waf.shape),
                  _full(wap.shape), _full(wcf.shape), _full(wcp.shape),
                  _full(bc.shape)],
        out_specs=[_rows(bs, K), _rows(bs, K)],
        out_shape=[jax.ShapeDtypeStruct((NP, K), f32)] * 2,
    )(feats, pts8, waf, wap, wcf, wcp, bc)


def _tc_msg1(E, ta, tb, w2, b2, w3, b3):
    bs = 4096

    def body(ra, rb, rw2, rb2, rw3, rb3, o):
        t = jax.nn.relu(ra[...] + rb[...])
        m = jax.nn.relu(_dot(t, rw2[...]) + rb2[...])
        o[...] = jax.nn.relu(_dot(m, rw3[...]) + rb3[...])

    return _pc(
        body,
        grid=(E // bs,),
        in_specs=[_rows(bs, 16), _rows(bs, 16), _full(w2.shape),
                  _full(b2.shape), _full(w3.shape), _full(b3.shape)],
        out_specs=_rows(bs, 64),
        out_shape=jax.ShapeDtypeStruct((E, 64), f32),
    )(ta, tb, w2, b2, w3, b3)


def _tc_msg2(E, K, ta, tb, w2, b2):
    bs = 2048 if K == 64 else 1024

    def body(ra, rb, rw2, rb2, o):
        t = jax.nn.relu(ra[...] + rb[...])
        o[...] = jax.nn.relu(_dot(t, rw2[...]) + rb2[...])

    return _pc(
        body,
        grid=(E // bs,),
        in_specs=[_rows(bs, K), _rows(bs, K), _full(w2.shape),
                  _full(b2.shape)],
        out_specs=_rows(bs, K),
        out_shape=jax.ShapeDtypeStruct((E, K), f32),
    )(ta, tb, w2, b2)


def _tc_msg2b(E, ta, tb, w2, b2):
    bs = 2048

    def body(ra, rb, rw2, rb2, o):
        xa = jnp.concatenate([ra[f] for f in range(8)], axis=-1)
        xb = jnp.concatenate([rb[f] for f in range(8)], axis=-1)
        t = jax.nn.relu(xa + xb)
        o[...] = jax.nn.relu(_dot(t, rw2[...]) + rb2[...])

    blk = pl.BlockSpec((8, bs, 8), lambda i: (0, i, 0))
    return _pc(
        body,
        grid=(E // bs,),
        in_specs=[blk, blk, _full(w2.shape), _full(b2.shape)],
        out_specs=_rows(bs, 64),
        out_shape=jax.ShapeDtypeStruct((E, 64), f32),
    )(ta, tb, w2, b2)


def _tc_h1(feats, pts8, s0, s1, c0, c1, woh, woa, bo1, wo2, bo2,
           w2s, w2d, w2p, b21):
    bs = 512

    def body(rf, rp, rs0, rs1, rc0, rc1, rwoh, rwoa, rbo1, rwo2, rbo2,
             rw2s, rw2d, rw2p, rb21, oh, oa, oc):
        cnt = jnp.maximum((rc0[...] + rc1[...])[:, 0:1], 1.0)
        agg = (rs0[...] + rs1[...]) / cnt
        x = jax.nn.relu(_dot(rf[...], rwoh[...]) + _dot(agg, rwoa[...])
                        + rbo1[...])
        h = jax.nn.relu(_dot(x, rwo2[...]) + rbo2[...])
        oh[...] = h
        pp = _dot(rp[...], rw2p[...])
        av = _dot(h, rw2s[...]) + pp
        cv = _dot(h, rw2d[...]) - pp + rb21[...]
        oa[...] = jnp.stack([av[:, 8 * f:8 * f + 8] for f in range(8)])
        oc[...] = jnp.stack([cv[:, 8 * f:8 * f + 8] for f in range(8)])

    return _pc(
        body,
        grid=(NP // bs,),
        in_specs=[_rows(bs, 128), _rows(bs, 8), _rows(bs, 64), _rows(bs, 64),
                  _rows(bs, 16), _rows(bs, 16)] +
                 [_full(w.shape) for w in
                  (woh, woa, bo1, wo2, bo2, w2s, w2d, w2p, b21)],
        out_specs=[_rows(bs, 64),
                   pl.BlockSpec((8, bs, 8), lambda i: (0, i, 0)),
                   pl.BlockSpec((8, bs, 8), lambda i: (0, i, 0))],
        out_shape=[jax.ShapeDtypeStruct((NP, 64), f32),
                   jax.ShapeDtypeStruct((8, NP, 8), f32),
                   jax.ShapeDtypeStruct((8, NP, 8), f32)],
    )(feats, pts8, s0, s1, c0, c1, woh, woa, bo1, wo2, bo2,
      w2s, w2d, w2p, b21)


def _tc_h2(h1, s0, s1, c0, c1, woh, woa, bo1, wo2, bo2):
    bs = 512

    def body(rh, rs0, rs1, rc0, rc1, rwoh, rwoa, rbo1, rwo2, rbo2, o):
        cnt = jnp.maximum((rc0[...] + rc1[...])[:, 0:1], 1.0)
        agg = (rs0[...] + rs1[...]) / cnt
        x = jax.nn.relu(_dot(rh[...], rwoh[...]) + _dot(agg, rwoa[...])
                        + rbo1[...])
        o[...] = jax.nn.relu(_dot(x, rwo2[...]) + rbo2[...])

    return _pc(
        body,
        grid=(NP // bs,),
        in_specs=[_rows(bs, 64)] * 3 + [_rows(bs, 16)] * 2 +
                 [_full(w.shape) for w in (woh, woa, bo1, wo2, bo2)],
        out_specs=_rows(bs, 64),
        out_shape=jax.ShapeDtypeStruct((NP, 64), f32),
    )(h1, s0, s1, c0, c1, woh, woa, bo1, wo2, bo2)


def _tc_cluster1(sp0, sp1, cp0, cp1, cen8, w31a, w31c, b31, w32, b32,
                 wo41, bo41, wo42, bo42, w4s, w4d, w4p, b41):
    def body(rs0, rs1, rc0, rc1, rcen, rw31a, rw31c, rb31, rw32, rb32,
             rwo41, rbo41, rwo42, rbo42, rw4s, rw4d, rw4p, rb41,
             oh3, oa, oc):
        cnt = jnp.maximum((rc0[...] + rc1[...])[:, 0:1], 1.0)
        agg3 = (rs0[...] + rs1[...]) / cnt
        cen = rcen[...]
        x = jax.nn.relu(_dot(agg3, rw31a[...]) + _dot(cen, rw31c[...])
                        + rb31[...])
        h3 = jax.nn.relu(_dot(x, rw32[...]) + rb32[...])
        oh3[...] = h3
        o1 = jax.nn.relu(_dot(h3, rwo41[...]) + rbo41[...])
        off = _dot(o1, rwo42[...]) + rbo42[...]
        pos = cen + off
        pp = _dot(pos, rw4p[...])
        oa[...] = _dot(h3, rw4s[...]) + pp
        oc[...] = _dot(h3, rw4d[...]) - pp + rb41[...]

    ins = (sp0, sp1, cp0, cp1, cen8, w31a, w31c, b31, w32, b32,
           wo41, bo41, wo42, bo42, w4s, w4d, w4p, b41)
    return _pc(
        body,
        grid=(1,),
        in_specs=[_full(x.shape) for x in ins],
        out_specs=[_full((MP, 256)), _full((MP, 256)), _full((MP, 256))],
        out_shape=[jax.ShapeDtypeStruct((MP, 256), f32)] * 3,
    )(*ins)


def _tc_h4(m8, h_in, cen8, woh, woa, bo1, wo2, bo2,
           wo41, bo41, wo42, bo42, wns, wnd, wnp, bn1):
    def body(rm8, rh, rcen, rwoh, rwoa, rbo1, rwo2, rbo2,
             rwo41, rbo41, rwo42, rbo42, rwns, rwnd, rwnp, rbn1,
             oh, oa, oc):
        agg = jnp.max(rm8[...], axis=0)
        x = jax.nn.relu(_dot(rh[...], rwoh[...]) + _dot(agg, rwoa[...])
                        + rbo1[...])
        h = jax.nn.relu(_dot(x, rwo2[...]) + rbo2[...])
        oh[...] = h
        o1 = jax.nn.relu(_dot(h, rwo41[...]) + rbo41[...])
        off = _dot(o1, rwo42[...]) + rbo42[...]
        pos = rcen[...] + off
        pp = _dot(pos, rwnp[...])
        oa[...] = _dot(h, rwns[...]) + pp
        oc[...] = _dot(h, rwnd[...]) - pp + rbn1[...]

    ins = (m8, h_in, cen8, woh, woa, bo1, wo2, bo2,
           wo41, bo41, wo42, bo42, wns, wnd, wnp, bn1)
    return _pc(
        body,
        grid=(1,),
        in_specs=[_full(x.shape) for x in ins],
        out_specs=[_full((MP, 256)), _full((MP, 256)), _full((MP, 256))],
        out_shape=[jax.ShapeDtypeStruct((MP, 256), f32)] * 3,
    )(*ins)


def _tc_h4b(m8, h_in, woh, woa, bo1, wo2, bo2, w51, b51, w52, b52,
            w6s, w6d, b61, wo6h):
    def body(rm8, rh, rwoh, rwoa, rbo1, rwo2, rbo2, rw51, rb51,
             rw52, rb52, rw6s, rw6d, rb61, rwo6h, oa, oc, ou):
        agg = jnp.max(rm8[...], axis=0)
        x = jax.nn.relu(_dot(rh[...], rwoh[...]) + _dot(agg, rwoa[...])
                        + rbo1[...])
        h4b = jax.nn.relu(_dot(x, rwo2[...]) + rbo2[...])
        y = jax.nn.relu(_dot(h4b, rw51[...]) + rb51[...])
        h5m = jax.nn.relu(_dot(y, rw52[...]) + rb52[...])
        oa[...] = _dot(h5m, rw6s[...])
        oc[...] = _dot(h5m, rw6d[...]) + rb61[...]
        ou[...] = _dot(h5m, rwo6h[...])

    ins = (m8, h_in, woh, woa, bo1, wo2, bo2, w51, b51, w52, b52,
           w6s, w6d, b61, wo6h)
    return _pc(
        body,
        grid=(1,),
        in_specs=[_full(x.shape) for x in ins],
        out_specs=[_full((MP, 64))] * 3,
        out_shape=[jax.ShapeDtypeStruct((MP, 64), f32)] * 3,
    )(*ins)


def _tc_a6c6(A6, C6, pts8, w6p):
    bs = 512

    def body(ra, rc, rp, rw, oa, oc):
        pp = _dot(rp[...], rw[...])
        av = ra[...] + pp
        cv = rc[...] - pp
        oa[...] = jnp.stack([av[:, 8 * f:8 * f + 8] for f in range(8)])
        oc[...] = jnp.stack([cv[:, 8 * f:8 * f + 8] for f in range(8)])

    return _pc(
        body,
        grid=(NP // bs,),
        in_specs=[_rows(bs, 64), _rows(bs, 64), _rows(bs, 8),
                  _full(w6p.shape)],
        out_specs=[pl.BlockSpec((8, bs, 8), lambda i: (0, i, 0))] * 2,
        out_shape=[jax.ShapeDtypeStruct((8, NP, 8), f32)] * 2,
    )(A6, C6, pts8, w6p)


def _tc_h6(U6, s0, s1, c0, c1, h2, woa, bo1, wo2, bo2, wc, bc):
    bs = 512

    def body(ru, rs0, rs1, rc0, rc1, rh2, rwoa, rbo1, rwo2, rbo2,
             rwc, rbc, o):
        cnt = jnp.maximum((rc0[...] + rc1[...])[:, 0:1], 1.0)
        agg = (rs0[...] + rs1[...]) / cnt
        x = jax.nn.relu(ru[...] + _dot(agg, rwoa[...]) + rbo1[...])
        h6 = jax.nn.relu(_dot(x, rwo2[...]) + rbo2[...])
        fin = h6 + rh2[...]
        o[...] = _dot(fin, rwc[...]) + rbc[...]

    return _pc(
        body,
        grid=(NP // bs,),
        in_specs=[_rows(bs, 64)] * 3 + [_rows(bs, 16)] * 2 +
                 [_rows(bs, 64)] +
                 [_full(w.shape) for w in (woa, bo1, wo2, bo2, wc, bc)],
        out_specs=_rows(bs, 16),
        out_shape=jax.ShapeDtypeStruct((NP, 16), f32),
    )(U6, s0, s1, c0, c1, h2, woa, bo1, wo2, bo2, wc, bc)


def _split_edge_w(W1, D):
    ws, wd, wp = W1[:D], W1[D:2 * D], W1[2 * D:]
    wp8 = jnp.pad(wp, ((0, 8 - wp.shape[0]), (0, 0)))
    return ws, wd, wp8


def _row(b):
    return b.reshape(1, -1)


def _pad_idx(idx, E, fill, C):
    pad = jnp.full((E - idx.shape[0],), fill, i32)
    return jnp.concatenate([idx, pad]).reshape(E // C, C)


def kernel(features, points, cluster_centers, l0_edges, l1_edges, labels,
           params):
    p = params

    feats = jnp.pad(features, ((0, NP - N), (0, 0)))
    pts8 = jnp.pad(points, ((0, NP - N), (0, 5)))
    cen8 = jnp.pad(cluster_centers, ((0, MP - M), (0, 5)))

    src0 = _pad_idx(l0_edges[0], E0P, 0, 128)
    dst0 = _pad_idx(l0_edges[1], E0P, NP - 1, 128)
    src1 = _pad_idx(l1_edges[0], E1P, 0, 64)
    dst1 = _pad_idx(l1_edges[1], E1P, MP - 1, 64)
    dst1f = dst1.reshape(E1P)
    lab64 = _pad_idx(labels, NP, MP - 1, 64)
    lab32 = lab64.reshape(NP // 32, 32)

    zN64 = jnp.zeros((NC, NP, 64), f32)
    zN16 = jnp.zeros((NC, NP, 16), f32)
    zM64 = jnp.zeros((NC, MP, 64), f32)
    zM16 = jnp.zeros((NC, MP, 16), f32)
    zm64 = jnp.zeros((MP, 64), f32)
    ones128 = jnp.ones((4, 128, 16), f32)
    ones64 = jnp.ones((5, 64, 16), f32)
    E0H = E0P // 2
    srcH = (src0[:E0H // 128], src0[E0H // 128:])
    dstH = (dst0[:E0H // 128], dst0[E0H // 128:])
    src0f = src0.reshape(E0P)
    dst0f = dst0.reshape(E0P)
    srcHf = (src0f[:E0H], src0f[E0H:])
    dstHf = (dst0f[:E0H], dst0f[E0H:])

    (W11, b11), (W12, b12), (W13, b13) = p['l1_edge']
    ws, wd, wp8 = _split_edge_w(W11, 128)
    a1, c1 = _tc_node_proj(feats, pts8, ws, wp8, wd, -wp8, _row(b11))
    s1, cnt0 = zN64, zN16
    for hi in range(2):
        ta, tb = _sc_gather(2, NP, E0H, 128, 16, 10)(a1, c1, srcH[hi],
                                                    dstH[hi])
        msg = _tc_msg1(E0H, ta.reshape(E0H, 16), tb.reshape(E0H, 16),
                       W12, _row(b12), W13, _row(b13))
        s1, cnt0 = _sc_scatter_add(E0H, 128, NP, True, 4)(
            msg.reshape(E0H // 128, 128, 64), dstH[hi], s1, cnt0, ones128)

    (Wo11, bo11), (Wo12, bo12) = p['l1_out']
    (W21, b21) = p['l2_edge'][0]
    w2s, w2d, w2p8 = _split_edge_w(W21, 64)
    h1, a2, c2 = _tc_h1(
        feats, pts8, s1[0], s1[1], cnt0[0], cnt0[1],
        Wo11[:128], Wo11[128:], _row(bo11), Wo12, _row(bo12),
        w2s, w2d, w2p8, _row(b21))

    (W22, b22) = p['l2_edge'][1]
    s2 = zN64
    for hi in range(2):
        ta, tb = _sc_vgather2(E0H, 2048)(a2, srcHf[hi], c2, dstHf[hi])
        msg = _tc_msg2b(E0H, ta, tb, W22, _row(b22))
        (s2,) = _sc_scatter_add(E0H, 128, NP, False, 4)(
            msg.reshape(E0H // 128, 128, 64), dstH[hi], s2)
    (Wo21, bo21), (Wo22, bo22) = p['l2_out']
    h2 = _tc_h2(h1, s2[0], s2[1], cnt0[0], cnt0[1],
                Wo21[:64], Wo21[64:], _row(bo21), Wo22, _row(bo22))

    sp, cntp = _sc_scatter_add(NP, 64, MP, True, 5)(
        h2.reshape(NP // 64, 64, 64), lab64, zM64, zM16, ones64)

    (W31, b31), (W32, b32) = p['l3_out']
    (W4o1, b4o1), (W4o2, b4o2) = p['l4_off']
    (W41, b41) = p['l4_edge'][0]
    w4s, w4d, w4p8 = _split_edge_w(W41, 256)
    w4o2_8 = jnp.pad(W4o2, ((0, 0), (0, 5)))
    b4o2_8 = jnp.pad(b4o2, ((0, 5),))
    w31c8 = jnp.pad(W31[64:], ((0, 5), (0, 0)))
    h3, a4, c4 = _tc_cluster1(
        sp[0], sp[1], cntp[0], cntp[1], cen8,
        W31[:64], w31c8, _row(b31), W32, _row(b32),
        W4o1, _row(b4o1), w4o2_8, _row(b4o2_8),
        w4s, w4d, w4p8, _row(b41))

    (W42, b42) = p['l4_edge'][1]
    t4a, t4b = _sc_gather(2, MP, E1P, 64, 256, 2)(a4, c4, src1, dst1)
    msg4 = _tc_msg2(E1P, 256, t4a.reshape(E1P, 256), t4b.reshape(E1P, 256),
                    W42, _row(b42))
    (m4,) = _sc_scatter_max(E1P, MP, 256)(msg4, dst1f, zm64)

    (Wo41, bo41), (Wo42, bo42) = p['l4_out']
    (W4bo1, b4bo1), (W4bo2, b4bo2) = p['l4b_off']
    (W4b1, b4b1) = p['l4b_edge'][0]
    w4bs, w4bd, w4bp8 = _split_edge_w(W4b1, 256)
    w4bo2_8 = jnp.pad(W4bo2, ((0, 0), (0, 5)))
    b4bo2_8 = jnp.pad(b4bo2, ((0, 5),))
    h4, a4b, c4b = _tc_h4(
        m4, h3, cen8,
        Wo41[:256], Wo41[256:], _row(bo41), Wo42, _row(bo42),
        W4bo1, _row(b4bo1), w4bo2_8, _row(b4bo2_8),
        w4bs, w4bd, w4bp8, _row(b4b1))

    (W4b2, b4b2) = p['l4b_edge'][1]
    t4ba, t4bb = _sc_gather(2, MP, E1P, 64, 256, 2)(a4b, c4b, src1, dst1)
    msg4b = _tc_msg2(E1P, 256, t4ba.reshape(E1P, 256),
                     t4bb.reshape(E1P, 256), W4b2, _row(b4b2))
    (m4b,) = _sc_scatter_max(E1P, MP, 256)(msg4b, dst1f, zm64)

    (Wo4b1, bo4b1), (Wo4b2, bo4b2) = p['l4b_out']
    (W51, b51), (W52, b52) = p['l5_out']
    (W61, b61) = p['l6_edge'][0]
    w6s, w6d, w6p8 = _split_edge_w(W61, 64)
    (Wo61, bo61), (Wo62, bo62) = p['l6_out']
    a6m, c6m, u6m = _tc_h4b(
        m4b, h4,
        Wo4b1[:256], Wo4b1[256:], _row(bo4b1), Wo4b2, _row(bo4b2),
        W51, _row(b51), W52, _row(b52),
        w6s, w6d, _row(b61), Wo61[:64])

    A6, C6, U6 = _sc_gather(3, MP, NP, 32, 64, 5)(
        a6m, c6m, u6m, lab32, lab32, lab32)
    A6, C6, U6 = (A6.reshape(NP, 64), C6.reshape(NP, 64),
                  U6.reshape(NP, 64))
    a6, c6 = _tc_a6c6(A6, C6, pts8, w6p8)
    (W62, b62) = p['l6_edge'][1]
    s6 = zN64
    for hi in range(2):
        ta, tb = _sc_vgather2(E0H, 2048)(a6, srcHf[hi], c6, dstHf[hi])
        msg = _tc_msg2b(E0H, ta, tb, W62, _row(b62))
        (s6,) = _sc_scatter_add(E0H, 128, NP, False, 4)(
            msg.reshape(E0H // 128, 128, 64), dstH[hi], s6)

    Wc, bc = p['cls'][0]
    out = _tc_h6(U6, s6[0], s6[1], cnt0[0], cnt0[1], h2,
                 Wo61[64:], _row(bo61), Wo62, _row(bo62), Wc, _row(bc))
    return out[:N]


# speedup vs baseline: 2.3680x; 2.3680x over previous
import functools

import jax
import jax.numpy as jnp
from jax import lax
from jax.experimental import pallas as pl
from jax.experimental.pallas import tpu as pltpu
from jax.experimental.pallas import tpu_sc as plsc

_pc = pl.pallas_call

N = 10000
M = 1000
E0 = 320000
E1 = 16000
NP = 10240
MP = 1024
E0P = 327680
E1P = 16384
NC, NS = 2, 16
NW = NC * NS

def _mesh():
    return plsc.VectorSubcoreMesh(core_axis_name="c", subcore_axis_name="s")


_SC_PARAMS = pltpu.CompilerParams(use_tc_tiling_on_sc=False)
_SC_PARAMS_NL = pltpu.CompilerParams(use_tc_tiling_on_sc=False,
                                     needs_layout_passes=False)


f32 = jnp.float32
i32 = jnp.int32


def _sc_gather(n_t, n_rows, E, C, K, G):
    per_w = E // NW
    n_ch = per_w // C
    n_g = n_ch // G
    assert n_g * G == n_ch

    def body(*refs):
        tabs = refs[:n_t]
        idxs = refs[n_t:2 * n_t]
        outs = refs[2 * n_t:3 * n_t]
        idx_v, buf0, buf1, sem0, sem1 = refs[3 * n_t:]
        bufs = (buf0, buf1)
        sems = (sem0, sem1)
        wid = lax.axis_index("s") * NC + lax.axis_index("c")
        row0 = wid * n_ch

        for t in range(n_t):
            pltpu.sync_copy(idxs[t].at[pl.ds(row0, n_ch)], idx_v)

            def fire(g, b, t=t):
                return [pltpu.async_copy(
                    tabs[t].at[idx_v.at[g * G + i]],
                    bufs[b].at[i], sems[b]) for i in range(G)]

            pending = fire(0, 0)
            for g in range(n_g):
                b = g % 2
                nxt = fire(g + 1, 1 - b) if g + 1 < n_g else []
                for h in pending:
                    h.wait()
                pltpu.sync_copy(
                    bufs[b],
                    outs[t].at[pl.ds(row0 + g * G, G)])
                pending = nxt

    return pl.kernel(
        body,
        out_type=[jax.ShapeDtypeStruct((E // C, C, K), f32)
                  for _ in range(n_t)],
        mesh=_mesh(), compiler_params=_SC_PARAMS,
        name=f"scg_{n_t}x{E}x{K}",
        scratch_types=[
            pltpu.VMEM((n_ch, C), i32),
            pltpu.VMEM((G, C, K), f32),
            pltpu.VMEM((G, C, K), f32),
            pltpu.SemaphoreType.DMA,
            pltpu.SemaphoreType.DMA,
        ],
    )


def _sc_scatter_add(E, C, Np, with_count, G):
    per_w = E // NW
    n_ch = per_w // C
    n_g = n_ch // G
    assert n_g * G == n_ch
    rpt = Np // NS

    def body(*refs):
        if with_count:
            (msg, idx, init, initc, ones, out, cnt_out,
             acc, cacc, idx_v, msg0, msg1, ones_v, sem0, sem1) = refs
        else:
            (msg, idx, init, out, acc, idx_v, msg0, msg1,
             sem0, sem1) = refs
        msgs = (msg0, msg1)
        sems = (sem0, sem1)
        c = lax.axis_index("c")
        s = lax.axis_index("s")
        wid = s * NC + c
        pltpu.sync_copy(init.at[c, pl.ds(s * rpt, rpt)],
                        acc.at[pl.ds(s * rpt, rpt)])
        if with_count:
            pltpu.sync_copy(initc.at[c, pl.ds(s * rpt, rpt)],
                            cacc.at[pl.ds(s * rpt, rpt)])
            pltpu.sync_copy(ones, ones_v)
        plsc.subcore_barrier()
        pltpu.sync_copy(idx.at[pl.ds(wid * n_ch, n_ch)], idx_v)

        def fire(g, b):
            hs = []
            for i in range(G):
                hs.append(pltpu.async_copy(
                    msgs[b].at[i], acc.at[idx_v.at[g * G + i]],
                    sems[b], add=True))
                if with_count:
                    hs.append(pltpu.async_copy(
                        ones_v.at[i], cacc.at[idx_v.at[g * G + i]],
                        sems[b], add=True))
            return hs

        pltpu.sync_copy(msg.at[pl.ds(wid * n_ch, G)], msg0)
        pending = fire(0, 0)
        for g in range(n_g):
            b = g % 2
            if g + 1 < n_g:
                pltpu.sync_copy(
                    msg.at[pl.ds(wid * n_ch + (g + 1) * G, G)],
                    msgs[1 - b])
                nxt = fire(g + 1, 1 - b)
            else:
                nxt = []
            for h in pending:
                h.wait()
            pending = nxt
        plsc.subcore_barrier()
        pltpu.sync_copy(acc.at[pl.ds(s * rpt, rpt)],
                        out.at[c, pl.ds(s * rpt, rpt)])
        if with_count:
            pltpu.sync_copy(cacc.at[pl.ds(s * rpt, rpt)],
                            cnt_out.at[c, pl.ds(s * rpt, rpt)])

    out_type = [jax.ShapeDtypeStruct((NC, Np, 64), f32)]
    scratch = [
        pltpu.VMEM_SHARED((Np, 64), f32),
    ]
    if with_count:
        out_type.append(jax.ShapeDtypeStruct((NC, Np, 16), f32))
        scratch.append(pltpu.VMEM_SHARED((Np, 16), f32))
    scratch += [pltpu.VMEM((n_ch, C), i32),
                pltpu.VMEM((G, C, 64), f32),
                pltpu.VMEM((G, C, 64), f32)]
    if with_count:
        scratch.append(pltpu.VMEM((G, C, 16), f32))
    scratch += [pltpu.SemaphoreType.DMA, pltpu.SemaphoreType.DMA]
    return pl.kernel(body, out_type=out_type, mesh=_mesh(),
                     compiler_params=_SC_PARAMS,
                     name=f"sca_{E}x{Np}" + ("_cnt" if with_count else ""),
                     scratch_types=scratch)


def _sc_scatter_max(E, Np, K):
    NEG = 8
    NFG = 4
    per_g = E // NEG
    CH = 512
    n_ch = per_g // CH

    def body(msg, idx, z64, out, acc, idx_v, buf):
        c = lax.axis_index("c")
        s = lax.axis_index("s")
        eg = c * (NS // NFG) + s // NFG
        fg = s % NFG
        lanes = [lax.iota(i32, 16) + 16 * q for q in range(4)]
        pltpu.sync_copy(z64, acc)
        pltpu.sync_copy(idx.at[pl.ds(eg * per_g, per_g)], idx_v)
        for k in range(n_ch):
            pltpu.sync_copy(
                msg.at[pl.ds(eg * per_g + k * CH, CH), pl.ds(fg * 64, 64)],
                buf)

            def step(e, _, k=k):
                dstv = plsc.load_gather(idx_v, [jnp.full((16,), k * CH + e, i32)])
                ev = jnp.full((16,), e, i32)
                for q in range(4):
                    val = plsc.load_gather(buf, [ev, lanes[q]])
                    cur = plsc.load_gather(acc, [dstv, lanes[q]])
                    plsc.store_scatter(acc, [dstv, lanes[q]],
                                       jnp.maximum(cur, val))
                return 0

            lax.fori_loop(0, CH, step, 0)
        pltpu.sync_copy(acc, out.at[eg, pl.ds(0, Np), pl.ds(fg * 64, 64)])

    return pl.kernel(
        body,
        out_type=[jax.ShapeDtypeStruct((NEG, Np, K), f32)],
        mesh=_mesh(), compiler_params=_SC_PARAMS_NL,
        name=f"scm_{E}x{Np}",
        scratch_types=[
            pltpu.VMEM((Np, 64), f32),
            pltpu.VMEM((per_g,), i32),
            pltpu.VMEM((CH, 64), f32),
        ],
    )


def _dot(a, b):
    return jnp.dot(a, b, preferred_element_type=f32)


def _rows(bs, w):
    return pl.BlockSpec((bs, w), lambda i: (i, 0))


def _full(shape):
    return pl.BlockSpec(shape, lambda i: tuple(0 for _ in shape))


def _tc_node_proj(feats, pts8, waf, wap, wcf, wcp, bc):
    K = waf.shape[1]
    bs = 512

    def body(xf, xp, rwaf, rwap, rwcf, rwcp, rbc, oa, oc):
        fb = xf[...]
        pb = xp[...]
        oa[...] = _dot(fb, rwaf[...]) + _dot(pb, rwap[...])
        oc[...] = _dot(fb, rwcf[...]) + _dot(pb, rwcp[...]) + rbc[...]

    return _pc(
        body,
        grid=(NP // bs,),
        in_specs=[_rows(bs, 128), _rows(bs, 8), _full(waf.shape),
                  _full(wap.shape), _full(wcf.shape), _full(wcp.shape),
                  _full(bc.shape)],
        out_specs=[_rows(bs, K), _rows(bs, K)],
        out_shape=[jax.ShapeDtypeStruct((NP, K), f32)] * 2,
    )(feats, pts8, waf, wap, wcf, wcp, bc)


def _tc_msg1(E, ta, tb, w2, b2, w3, b3):
    bs = 4096

    def body(ra, rb, rw2, rb2, rw3, rb3, o):
        t = jax.nn.relu(ra[...] + rb[...])
        m = jax.nn.relu(_dot(t, rw2[...]) + rb2[...])
        o[...] = jax.nn.relu(_dot(m, rw3[...]) + rb3[...])

    return _pc(
        body,
        grid=(E // bs,),
        in_specs=[_rows(bs, 16), _rows(bs, 16), _full(w2.shape),
                  _full(b2.shape), _full(w3.shape), _full(b3.shape)],
        out_specs=_rows(bs, 64),
        out_shape=jax.ShapeDtypeStruct((E, 64), f32),
    )(ta, tb, w2, b2, w3, b3)


def _tc_msg2(E, K, ta, tb, w2, b2):
    bs = 2048 if K == 64 else 1024

    def body(ra, rb, rw2, rb2, o):
        t = jax.nn.relu(ra[...] + rb[...])
        o[...] = jax.nn.relu(_dot(t, rw2[...]) + rb2[...])

    return _pc(
        body,
        grid=(E // bs,),
        in_specs=[_rows(bs, K), _rows(bs, K), _full(w2.shape),
                  _full(b2.shape)],
        out_specs=_rows(bs, K),
        out_shape=jax.ShapeDtypeStruct((E, K), f32),
    )(ta, tb, w2, b2)


def _tc_h1(feats, pts8, s0, s1, c0, c1, woh, woa, bo1, wo2, bo2,
           w2s, w2d, w2p, b21):
    bs = 512

    def body(rf, rp, rs0, rs1, rc0, rc1, rwoh, rwoa, rbo1, rwo2, rbo2,
             rw2s, rw2d, rw2p, rb21, oh, oa, oc):
        cnt = jnp.maximum((rc0[...] + rc1[...])[:, 0:1], 1.0)
        agg = (rs0[...] + rs1[...]) / cnt
        x = jax.nn.relu(_dot(rf[...], rwoh[...]) + _dot(agg, rwoa[...])
                        + rbo1[...])
        h = jax.nn.relu(_dot(x, rwo2[...]) + rbo2[...])
        oh[...] = h
        pp = _dot(rp[...], rw2p[...])
        oa[...] = _dot(h, rw2s[...]) + pp
        oc[...] = _dot(h, rw2d[...]) - pp + rb21[...]

    return _pc(
        body,
        grid=(NP // bs,),
        in_specs=[_rows(bs, 128), _rows(bs, 8), _rows(bs, 64), _rows(bs, 64),
                  _rows(bs, 16), _rows(bs, 16)] +
                 [_full(w.shape) for w in
                  (woh, woa, bo1, wo2, bo2, w2s, w2d, w2p, b21)],
        out_specs=[_rows(bs, 64)] * 3,
        out_shape=[jax.ShapeDtypeStruct((NP, 64), f32)] * 3,
    )(feats, pts8, s0, s1, c0, c1, woh, woa, bo1, wo2, bo2,
      w2s, w2d, w2p, b21)


def _tc_h2(h1, s0, s1, c0, c1, woh, woa, bo1, wo2, bo2):
    bs = 512

    def body(rh, rs0, rs1, rc0, rc1, rwoh, rwoa, rbo1, rwo2, rbo2, o):
        cnt = jnp.maximum((rc0[...] + rc1[...])[:, 0:1], 1.0)
        agg = (rs0[...] + rs1[...]) / cnt
        x = jax.nn.relu(_dot(rh[...], rwoh[...]) + _dot(agg, rwoa[...])
                        + rbo1[...])
        o[...] = jax.nn.relu(_dot(x, rwo2[...]) + rbo2[...])

    return _pc(
        body,
        grid=(NP // bs,),
        in_specs=[_rows(bs, 64)] * 3 + [_rows(bs, 16)] * 2 +
                 [_full(w.shape) for w in (woh, woa, bo1, wo2, bo2)],
        out_specs=_rows(bs, 64),
        out_shape=jax.ShapeDtypeStruct((NP, 64), f32),
    )(h1, s0, s1, c0, c1, woh, woa, bo1, wo2, bo2)


def _tc_cluster1(sp0, sp1, cp0, cp1, cen8, w31a, w31c, b31, w32, b32,
                 wo41, bo41, wo42, bo42, w4s, w4d, w4p, b41):
    def body(rs0, rs1, rc0, rc1, rcen, rw31a, rw31c, rb31, rw32, rb32,
             rwo41, rbo41, rwo42, rbo42, rw4s, rw4d, rw4p, rb41,
             oh3, oa, oc):
        cnt = jnp.maximum((rc0[...] + rc1[...])[:, 0:1], 1.0)
        agg3 = (rs0[...] + rs1[...]) / cnt
        cen = rcen[...]
        x = jax.nn.relu(_dot(agg3, rw31a[...]) + _dot(cen, rw31c[...])
                        + rb31[...])
        h3 = jax.nn.relu(_dot(x, rw32[...]) + rb32[...])
        oh3[...] = h3
        o1 = jax.nn.relu(_dot(h3, rwo41[...]) + rbo41[...])
        off = _dot(o1, rwo42[...]) + rbo42[...]
        pos = cen + off
        pp = _dot(pos, rw4p[...])
        oa[...] = _dot(h3, rw4s[...]) + pp
        oc[...] = _dot(h3, rw4d[...]) - pp + rb41[...]

    ins = (sp0, sp1, cp0, cp1, cen8, w31a, w31c, b31, w32, b32,
           wo41, bo41, wo42, bo42, w4s, w4d, w4p, b41)
    return _pc(
        body,
        grid=(1,),
        in_specs=[_full(x.shape) for x in ins],
        out_specs=[_full((MP, 256)), _full((MP, 256)), _full((MP, 256))],
        out_shape=[jax.ShapeDtypeStruct((MP, 256), f32)] * 3,
    )(*ins)


def _tc_h4(m8, h_in, cen8, woh, woa, bo1, wo2, bo2,
           wo41, bo41, wo42, bo42, wns, wnd, wnp, bn1):
    def body(rm8, rh, rcen, rwoh, rwoa, rbo1, rwo2, rbo2,
             rwo41, rbo41, rwo42, rbo42, rwns, rwnd, rwnp, rbn1,
             oh, oa, oc):
        agg = jnp.max(rm8[...], axis=0)
        x = jax.nn.relu(_dot(rh[...], rwoh[...]) + _dot(agg, rwoa[...])
                        + rbo1[...])
        h = jax.nn.relu(_dot(x, rwo2[...]) + rbo2[...])
        oh[...] = h
        o1 = jax.nn.relu(_dot(h, rwo41[...]) + rbo41[...])
        off = _dot(o1, rwo42[...]) + rbo42[...]
        pos = rcen[...] + off
        pp = _dot(pos, rwnp[...])
        oa[...] = _dot(h, rwns[...]) + pp
        oc[...] = _dot(h, rwnd[...]) - pp + rbn1[...]

    ins = (m8, h_in, cen8, woh, woa, bo1, wo2, bo2,
           wo41, bo41, wo42, bo42, wns, wnd, wnp, bn1)
    return _pc(
        body,
        grid=(1,),
        in_specs=[_full(x.shape) for x in ins],
        out_specs=[_full((MP, 256)), _full((MP, 256)), _full((MP, 256))],
        out_shape=[jax.ShapeDtypeStruct((MP, 256), f32)] * 3,
    )(*ins)


def _tc_h4b(m8, h_in, woh, woa, bo1, wo2, bo2, w51, b51, w52, b52,
            w6s, w6d, b61, wo6h):
    def body(rm8, rh, rwoh, rwoa, rbo1, rwo2, rbo2, rw51, rb51,
             rw52, rb52, rw6s, rw6d, rb61, rwo6h, oa, oc, ou):
        agg = jnp.max(rm8[...], axis=0)
        x = jax.nn.relu(_dot(rh[...], rwoh[...]) + _dot(agg, rwoa[...])
                        + rbo1[...])
        h4b = jax.nn.relu(_dot(x, rwo2[...]) + rbo2[...])
        y = jax.nn.relu(_dot(h4b, rw51[...]) + rb51[...])
        h5m = jax.nn.relu(_dot(y, rw52[...]) + rb52[...])
        oa[...] = _dot(h5m, rw6s[...])
        oc[...] = _dot(h5m, rw6d[...]) + rb61[...]
        ou[...] = _dot(h5m, rwo6h[...])

    ins = (m8, h_in, woh, woa, bo1, wo2, bo2, w51, b51, w52, b52,
           w6s, w6d, b61, wo6h)
    return _pc(
        body,
        grid=(1,),
        in_specs=[_full(x.shape) for x in ins],
        out_specs=[_full((MP, 64))] * 3,
        out_shape=[jax.ShapeDtypeStruct((MP, 64), f32)] * 3,
    )(*ins)


def _tc_a6c6(A6, C6, pts8, w6p):
    bs = 512

    def body(ra, rc, rp, rw, oa, oc):
        pp = _dot(rp[...], rw[...])
        oa[...] = ra[...] + pp
        oc[...] = rc[...] - pp

    return _pc(
        body,
        grid=(NP // bs,),
        in_specs=[_rows(bs, 64), _rows(bs, 64), _rows(bs, 8),
                  _full(w6p.shape)],
        out_specs=[_rows(bs, 64)] * 2,
        out_shape=[jax.ShapeDtypeStruct((NP, 64), f32)] * 2,
    )(A6, C6, pts8, w6p)


def _tc_h6(U6, s0, s1, c0, c1, h2, woa, bo1, wo2, bo2, wc, bc):
    bs = 512

    def body(ru, rs0, rs1, rc0, rc1, rh2, rwoa, rbo1, rwo2, rbo2,
             rwc, rbc, o):
        cnt = jnp.maximum((rc0[...] + rc1[...])[:, 0:1], 1.0)
        agg = (rs0[...] + rs1[...]) / cnt
        x = jax.nn.relu(ru[...] + _dot(agg, rwoa[...]) + rbo1[...])
        h6 = jax.nn.relu(_dot(x, rwo2[...]) + rbo2[...])
        fin = h6 + rh2[...]
        o[...] = _dot(fin, rwc[...]) + rbc[...]

    return _pc(
        body,
        grid=(NP // bs,),
        in_specs=[_rows(bs, 64)] * 3 + [_rows(bs, 16)] * 2 +
                 [_rows(bs, 64)] +
                 [_full(w.shape) for w in (woa, bo1, wo2, bo2, wc, bc)],
        out_specs=_rows(bs, 16),
        out_shape=jax.ShapeDtypeStruct((NP, 16), f32),
    )(U6, s0, s1, c0, c1, h2, woa, bo1, wo2, bo2, wc, bc)


def _split_edge_w(W1, D):
    ws, wd, wp = W1[:D], W1[D:2 * D], W1[2 * D:]
    wp8 = jnp.pad(wp, ((0, 8 - wp.shape[0]), (0, 0)))
    return ws, wd, wp8


def _row(b):
    return b.reshape(1, -1)


def _pad_idx(idx, E, fill, C):
    pad = jnp.full((E - idx.shape[0],), fill, i32)
    return jnp.concatenate([idx, pad]).reshape(E // C, C)


def kernel(features, points, cluster_centers, l0_edges, l1_edges, labels,
           params):
    p = params

    feats = jnp.pad(features, ((0, NP - N), (0, 0)))
    pts8 = jnp.pad(points, ((0, NP - N), (0, 5)))
    cen8 = jnp.pad(cluster_centers, ((0, MP - M), (0, 5)))

    src0 = _pad_idx(l0_edges[0], E0P, 0, 128)
    dst0 = _pad_idx(l0_edges[1], E0P, NP - 1, 128)
    src1 = _pad_idx(l1_edges[0], E1P, 0, 64)
    dst1 = _pad_idx(l1_edges[1], E1P, MP - 1, 64)
    dst1f = dst1.reshape(E1P)
    lab64 = _pad_idx(labels, NP, MP - 1, 64)
    lab32 = lab64.reshape(NP // 32, 32)

    zN64 = jnp.zeros((NC, NP, 64), f32)
    zN16 = jnp.zeros((NC, NP, 16), f32)
    zM64 = jnp.zeros((NC, MP, 64), f32)
    zM16 = jnp.zeros((NC, MP, 16), f32)
    zm64 = jnp.zeros((MP, 64), f32)
    ones128 = jnp.ones((4, 128, 16), f32)
    ones64 = jnp.ones((5, 64, 16), f32)
    E0H = E0P // 2
    srcH = (src0[:E0H // 128], src0[E0H // 128:])
    dstH = (dst0[:E0H // 128], dst0[E0H // 128:])

    (W11, b11), (W12, b12), (W13, b13) = p['l1_edge']
    ws, wd, wp8 = _split_edge_w(W11, 128)
    a1, c1 = _tc_node_proj(feats, pts8, ws, wp8, wd, -wp8, _row(b11))
    s1, cnt0 = zN64, zN16
    for hi in range(2):
        ta, tb = _sc_gather(2, NP, E0H, 128, 16, 10)(a1, c1, srcH[hi],
                                                    dstH[hi])
        msg = _tc_msg1(E0H, ta.reshape(E0H, 16), tb.reshape(E0H, 16),
                       W12, _row(b12), W13, _row(b13))
        s1, cnt0 = _sc_scatter_add(E0H, 128, NP, True, 4)(
            msg.reshape(E0H // 128, 128, 64), dstH[hi], s1, cnt0, ones128)

    (Wo11, bo11), (Wo12, bo12) = p['l1_out']
    (W21, b21) = p['l2_edge'][0]
    w2s, w2d, w2p8 = _split_edge_w(W21, 64)
    h1, a2, c2 = _tc_h1(
        feats, pts8, s1[0], s1[1], cnt0[0], cnt0[1],
        Wo11[:128], Wo11[128:], _row(bo11), Wo12, _row(bo12),
        w2s, w2d, w2p8, _row(b21))

    (W22, b22) = p['l2_edge'][1]
    s2 = zN64
    for hi in range(2):
        ta, tb = _sc_gather(2, NP, E0H, 128, 64, 5)(a2, c2, srcH[hi],
                                                    dstH[hi])
        msg = _tc_msg2(E0H, 64, ta.reshape(E0H, 64), tb.reshape(E0H, 64),
                       W22, _row(b22))
        (s2,) = _sc_scatter_add(E0H, 128, NP, False, 4)(
            msg.reshape(E0H // 128, 128, 64), dstH[hi], s2)
    (Wo21, bo21), (Wo22, bo22) = p['l2_out']
    h2 = _tc_h2(h1, s2[0], s2[1], cnt0[0], cnt0[1],
                Wo21[:64], Wo21[64:], _row(bo21), Wo22, _row(bo22))

    sp, cntp = _sc_scatter_add(NP, 64, MP, True, 5)(
        h2.reshape(NP // 64, 64, 64), lab64, zM64, zM16, ones64)

    (W31, b31), (W32, b32) = p['l3_out']
    (W4o1, b4o1), (W4o2, b4o2) = p['l4_off']
    (W41, b41) = p['l4_edge'][0]
    w4s, w4d, w4p8 = _split_edge_w(W41, 256)
    w4o2_8 = jnp.pad(W4o2, ((0, 0), (0, 5)))
    b4o2_8 = jnp.pad(b4o2, ((0, 5),))
    w31c8 = jnp.pad(W31[64:], ((0, 5), (0, 0)))
    h3, a4, c4 = _tc_cluster1(
        sp[0], sp[1], cntp[0], cntp[1], cen8,
        W31[:64], w31c8, _row(b31), W32, _row(b32),
        W4o1, _row(b4o1), w4o2_8, _row(b4o2_8),
        w4s, w4d, w4p8, _row(b41))

    (W42, b42) = p['l4_edge'][1]
    t4a, t4b = _sc_gather(2, MP, E1P, 64, 256, 2)(a4, c4, src1, dst1)
    msg4 = _tc_msg2(E1P, 256, t4a.reshape(E1P, 256), t4b.reshape(E1P, 256),
                    W42, _row(b42))
    (m4,) = _sc_scatter_max(E1P, MP, 256)(msg4, dst1f, zm64)

    (Wo41, bo41), (Wo42, bo42) = p['l4_out']
    (W4bo1, b4bo1), (W4bo2, b4bo2) = p['l4b_off']
    (W4b1, b4b1) = p['l4b_edge'][0]
    w4bs, w4bd, w4bp8 = _split_edge_w(W4b1, 256)
    w4bo2_8 = jnp.pad(W4bo2, ((0, 0), (0, 5)))
    b4bo2_8 = jnp.pad(b4bo2, ((0, 5),))
    h4, a4b, c4b = _tc_h4(
        m4, h3, cen8,
        Wo41[:256], Wo41[256:], _row(bo41), Wo42, _row(bo42),
        W4bo1, _row(b4bo1), w4bo2_8, _row(b4bo2_8),
        w4bs, w4bd, w4bp8, _row(b4b1))

    (W4b2, b4b2) = p['l4b_edge'][1]
    t4ba, t4bb = _sc_gather(2, MP, E1P, 64, 256, 2)(a4b, c4b, src1, dst1)
    msg4b = _tc_msg2(E1P, 256, t4ba.reshape(E1P, 256),
                     t4bb.reshape(E1P, 256), W4b2, _row(b4b2))
    (m4b,) = _sc_scatter_max(E1P, MP, 256)(msg4b, dst1f, zm64)

    (Wo4b1, bo4b1), (Wo4b2, bo4b2) = p['l4b_out']
    (W51, b51), (W52, b52) = p['l5_out']
    (W61, b61) = p['l6_edge'][0]
    w6s, w6d, w6p8 = _split_edge_w(W61, 64)
    (Wo61, bo61), (Wo62, bo62) = p['l6_out']
    a6m, c6m, u6m = _tc_h4b(
        m4b, h4,
        Wo4b1[:256], Wo4b1[256:], _row(bo4b1), Wo4b2, _row(bo4b2),
        W51, _row(b51), W52, _row(b52),
        w6s, w6d, _row(b61), Wo61[:64])

    A6, C6, U6 = _sc_gather(3, MP, NP, 32, 64, 5)(
        a6m, c6m, u6m, lab32, lab32, lab32)
    A6, C6, U6 = (A6.reshape(NP, 64), C6.reshape(NP, 64),
                  U6.reshape(NP, 64))
    a6, c6 = _tc_a6c6(A6, C6, pts8, w6p8)
    (W62, b62) = p['l6_edge'][1]
    s6 = zN64
    for hi in range(2):
        ta, tb = _sc_gather(2, NP, E0H, 128, 64, 5)(a6, c6, srcH[hi],
                                                    dstH[hi])
        msg = _tc_msg2(E0H, 64, ta.reshape(E0H, 64), tb.reshape(E0H, 64),
                       W62, _row(b62))
        (s6,) = _sc_scatter_add(E0H, 128, NP, False, 4)(
            msg.reshape(E0H // 128, 128, 64), dstH[hi], s6)

    Wc, bc = p['cls'][0]
    out = _tc_h6(U6, s6[0], s6[1], cnt0[0], cnt0[1], h2,
                 Wo61[64:], _row(bo61), Wo62, _row(bo62), Wc, _row(bc))
    return out[:N]
